# Initial kernel scaffold; baseline (speedup 1.0000x reference)
import jax
import jax.numpy as jnp
from jax.experimental import pallas as pl


def kernel(x, edge_index, W1_self, W1_neigh, b1, W2_self, W2_neigh, b2):
    raise NotImplementedError("write your pallas kernel here")


# speedup vs baseline: 1.0399x; 1.0399x over previous
import functools

import jax
import jax.numpy as jnp
from jax.experimental import pallas as pl
from jax.experimental.pallas import tpu as pltpu

N = 10000
D = 128
BN = 1000


def _dense_body(x_ref, n1_ref, w1s_ref, w1n_ref, b1_ref, w2s_ref, w2n_ref,
                b2_ref, hs_ref, p2_ref):
    x = x_ref[...]
    neigh = n1_ref[...]
    h = jnp.dot(x, w1s_ref[...], preferred_element_type=jnp.float32)
    h += jnp.dot(neigh, w1n_ref[...], preferred_element_type=jnp.float32)
    h = jnp.maximum(h + b1_ref[...], 0.0)
    hs_ref[...] = jnp.dot(h, w2s_ref[...], preferred_element_type=jnp.float32) + b2_ref[...]
    p2_ref[...] = jnp.dot(h, w2n_ref[...], preferred_element_type=jnp.float32)


def _dense(x, neigh1, W1_self, W1_neigh, b1, W2_self, W2_neigh, b2):
    grid = (N // BN,)
    return pl.pallas_call(
        _dense_body,
        grid=grid,
        in_specs=[
            pl.BlockSpec((BN, D), lambda i: (i, 0)),
            pl.BlockSpec((BN, D), lambda i: (i, 0)),
            pl.BlockSpec((D, D), lambda i: (0, 0)),
            pl.BlockSpec((D, D), lambda i: (0, 0)),
            pl.BlockSpec((1, D), lambda i: (0, 0)),
            pl.BlockSpec((D, 2), lambda i: (0, 0)),
            pl.BlockSpec((D, 2), lambda i: (0, 0)),
            pl.BlockSpec((1, 2), lambda i: (0, 0)),
        ],
        out_specs=[
            pl.BlockSpec((BN, 2), lambda i: (i, 0)),
            pl.BlockSpec((BN, 2), lambda i: (i, 0)),
        ],
        out_shape=[
            jax.ShapeDtypeStruct((N, 2), jnp.float32),
            jax.ShapeDtypeStruct((N, 2), jnp.float32),
        ],
    )(x, neigh1, W1_self, W1_neigh, b1.reshape(1, D), W2_self, W2_neigh,
      b2.reshape(1, 2))


def _combine_body(hs_ref, agg2_ref, deg_ref, out_ref):
    deg = jnp.maximum(deg_ref[...], 1.0)
    out_ref[...] = hs_ref[...] + agg2_ref[...] / deg


def _combine(hs, agg2, deg):
    grid = (N // BN,)
    return pl.pallas_call(
        _combine_body,
        grid=grid,
        in_specs=[
            pl.BlockSpec((BN, 2), lambda i: (i, 0)),
            pl.BlockSpec((BN, 2), lambda i: (i, 0)),
            pl.BlockSpec((BN, 1), lambda i: (i, 0)),
        ],
        out_specs=pl.BlockSpec((BN, 2), lambda i: (i, 0)),
        out_shape=jax.ShapeDtypeStruct((N, 2), jnp.float32),
    )(hs, agg2, deg)


def kernel(x, edge_index, W1_self, W1_neigh, b1, W2_self, W2_neigh, b2):
    src = edge_index[0].astype(jnp.int32)
    dst = edge_index[1].astype(jnp.int32)
    E = src.shape[0]

    deg = jax.ops.segment_sum(jnp.ones((E,), jnp.float32), dst, num_segments=N)
    degc = jnp.maximum(deg, 1.0)[:, None]

    agg1 = jax.ops.segment_sum(x[src], dst, num_segments=N)
    neigh1 = agg1 / degc

    hs, p2 = _dense(x, neigh1, W1_self, W1_neigh, b1, W2_self, W2_neigh, b2)

    agg2 = jax.ops.segment_sum(p2[src], dst, num_segments=N)
    return _combine(hs, agg2, deg.reshape(N, 1))


# speedup vs baseline: 6.7650x; 6.5055x over previous
import functools

import jax
import jax.numpy as jnp
from jax import lax
from jax.experimental import pallas as pl
from jax.experimental.pallas import tpu as pltpu
from jax.experimental.pallas import tpu_sc as plsc

N = 10000
D = 128
NP = 10240
RW2 = 16
NWORK = 32
CH = 128
K = 79
EPAD = NWORK * K * CH
BN = 1024
RPS = NP // 16


def _seg1_body(table, srcs, dsts, out, pdeg, src_v, dst_v, rows_v, zbuf,
               ones_v, acc, dacc, sem):
    cid = lax.axis_index("c")
    sid = lax.axis_index("s")
    wid = cid * 16 + sid

    pltpu.sync_copy(srcs.at[wid], src_v)
    pltpu.sync_copy(dsts.at[wid], dst_v)

    zero = jnp.zeros((16,), jnp.float32)
    one = jnp.ones((16,), jnp.float32)
    for r in range(16):
        for c in range(D // 16):
            zbuf[r, pl.ds(c * 16, 16)] = zero
    for c in range(CH // 16):
        ones_v[pl.ds(c * 16, 16)] = one
    base = sid * RPS

    def _zero_step(k, _):
        pltpu.sync_copy(zbuf, acc.at[pl.ds(base + k * 16, 16)])
        return 0

    lax.fori_loop(0, RPS // 16, _zero_step, 0)

    def _zero_deg(k, _):
        pltpu.sync_copy(zbuf.at[0], dacc.at[pl.ds(base + k * CH, CH)])
        return 0

    lax.fori_loop(0, RPS // CH, _zero_deg, 0)
    plsc.subcore_barrier()

    def _chunk(c, _):
        pltpu.async_copy(table.at[src_v.at[c]], rows_v, sem).wait()
        pltpu.sync_copy(rows_v, acc.at[dst_v.at[c]], add=True)
        pltpu.sync_copy(ones_v, dacc.at[dst_v.at[c]], add=True)
        return 0

    lax.fori_loop(0, K, _chunk, 0)
    plsc.subcore_barrier()

    pltpu.sync_copy(acc.at[pl.ds(base, RPS)], out.at[cid, pl.ds(base, RPS)])
    pltpu.sync_copy(dacc.at[pl.ds(base, RPS)], pdeg.at[cid, pl.ds(base, RPS)])


def _seg2_body(table, srcs, dsts, out, src_v, dst_v, rows_v, zbuf, acc, sem):
    cid = lax.axis_index("c")
    sid = lax.axis_index("s")
    wid = cid * 16 + sid

    pltpu.sync_copy(srcs.at[wid], src_v)
    pltpu.sync_copy(dsts.at[wid], dst_v)

    zero = jnp.zeros((16,), jnp.float32)
    for r in range(16):
        for c in range(RW2 // 16):
            zbuf[r, pl.ds(c * 16, 16)] = zero
    base = sid * RPS

    def _zero_step(k, _):
        pltpu.sync_copy(zbuf, acc.at[pl.ds(base + k * 16, 16)])
        return 0

    lax.fori_loop(0, RPS // 16, _zero_step, 0)
    plsc.subcore_barrier()

    def _chunk(c, _):
        pltpu.async_copy(table.at[src_v.at[c]], rows_v, sem).wait()
        pltpu.sync_copy(rows_v, acc.at[dst_v.at[c]], add=True)
        return 0

    lax.fori_loop(0, K, _chunk, 0)
    plsc.subcore_barrier()

    pltpu.sync_copy(acc.at[pl.ds(base, RPS)], out.at[cid, pl.ds(base, RPS)])


def _sc_seg1(table, srcs, dsts):
    mesh = plsc.VectorSubcoreMesh(core_axis_name="c", subcore_axis_name="s")
    f = pl.kernel(
        _seg1_body,
        out_type=(
            jax.ShapeDtypeStruct((2, NP, D), jnp.float32),
            jax.ShapeDtypeStruct((2, NP), jnp.float32),
        ),
        mesh=mesh,
        compiler_params=pltpu.CompilerParams(use_tc_tiling_on_sc=False),
        scratch_types=[
            pltpu.VMEM((K, CH), jnp.int32),
            pltpu.VMEM((K, CH), jnp.int32),
            pltpu.VMEM((CH, D), jnp.float32),
            pltpu.VMEM((16, D), jnp.float32),
            pltpu.VMEM((CH,), jnp.float32),
            pltpu.VMEM_SHARED((NP, D), jnp.float32),
            pltpu.VMEM_SHARED((NP,), jnp.float32),
            pltpu.SemaphoreType.DMA,
        ],
    )
    return f(table, srcs, dsts)


def _sc_seg2(table, srcs, dsts):
    mesh = plsc.VectorSubcoreMesh(core_axis_name="c", subcore_axis_name="s")
    f = pl.kernel(
        _seg2_body,
        out_type=jax.ShapeDtypeStruct((2, NP, RW2), jnp.float32),
        mesh=mesh,
        compiler_params=pltpu.CompilerParams(use_tc_tiling_on_sc=False),
        scratch_types=[
            pltpu.VMEM((K, CH), jnp.int32),
            pltpu.VMEM((K, CH), jnp.int32),
            pltpu.VMEM((CH, RW2), jnp.float32),
            pltpu.VMEM((16, RW2), jnp.float32),
            pltpu.VMEM_SHARED((NP, RW2), jnp.float32),
            pltpu.SemaphoreType.DMA,
        ],
    )
    return f(table, srcs, dsts)


def _dense_body(x_ref, p_ref, pd_ref, w1s_ref, w1n_ref, b1_ref, w2s_ref,
                w2n_ref, b2_ref, q_ref):
    x = x_ref[...]
    a = p_ref[0] + p_ref[1]
    pd = pd_ref[...]
    deg = pd[:, 0:1] + pd[:, 1:2]
    neigh = a / jnp.maximum(deg, 1.0)
    h = jnp.dot(x, w1s_ref[...], preferred_element_type=jnp.float32)
    h += jnp.dot(neigh, w1n_ref[...], preferred_element_type=jnp.float32)
    h = jnp.maximum(h + b1_ref[...], 0.0)
    p2 = jnp.dot(h, w2n_ref[...], preferred_element_type=jnp.float32)
    hs = jnp.dot(h, w2s_ref[...], preferred_element_type=jnp.float32) + b2_ref[...]
    q_ref[...] = jnp.concatenate(
        [p2, hs, deg, jnp.zeros((x.shape[0], RW2 - 5), jnp.float32)], axis=1)


def _dense(x_pad, part1, pdeg_t, W1_self, W1_neigh, b1, W2_self, W2_neigh, b2):
    grid = (NP // BN,)
    return pl.pallas_call(
        _dense_body,
        grid=grid,
        in_specs=[
            pl.BlockSpec((BN, D), lambda i: (i, 0)),
            pl.BlockSpec((2, BN, D), lambda i: (0, i, 0)),
            pl.BlockSpec((BN, 2), lambda i: (i, 0)),
            pl.BlockSpec((D, D), lambda i: (0, 0)),
            pl.BlockSpec((D, D), lambda i: (0, 0)),
            pl.BlockSpec((1, D), lambda i: (0, 0)),
            pl.BlockSpec((D, 2), lambda i: (0, 0)),
            pl.BlockSpec((D, 2), lambda i: (0, 0)),
            pl.BlockSpec((1, 2), lambda i: (0, 0)),
        ],
        out_specs=pl.BlockSpec((BN, RW2), lambda i: (i, 0)),
        out_shape=jax.ShapeDtypeStruct((NP, RW2), jnp.float32),
    )(x_pad, part1, pdeg_t, W1_self, W1_neigh, b1.reshape(1, D), W2_self,
      W2_neigh, b2.reshape(1, 2))


def _combine_body(q_ref, p2_ref, out_ref):
    q = q_ref[...]
    a = p2_ref[0] + p2_ref[1]
    deg = jnp.maximum(q[:, 4:5], 1.0)
    out_ref[...] = q[:, 2:4] + a[:, 0:2] / deg


def _combine(q, part2):
    grid = (NP // BN,)
    return pl.pallas_call(
        _combine_body,
        grid=grid,
        in_specs=[
            pl.BlockSpec((BN, RW2), lambda i: (i, 0)),
            pl.BlockSpec((2, BN, RW2), lambda i: (0, i, 0)),
        ],
        out_specs=pl.BlockSpec((BN, 2), lambda i: (i, 0)),
        out_shape=jax.ShapeDtypeStruct((NP, 2), jnp.float32),
    )(q, part2)


def kernel(x, edge_index, W1_self, W1_neigh, b1, W2_self, W2_neigh, b2):
    src = edge_index[0].astype(jnp.int32)
    dst = edge_index[1].astype(jnp.int32)
    E = src.shape[0]

    srcs = jnp.full((EPAD,), 0, jnp.int32).at[:E].set(src).reshape(NWORK, K, CH)
    dsts = jnp.full((EPAD,), N, jnp.int32).at[:E].set(dst).reshape(NWORK, K, CH)

    x_pad = jnp.zeros((NP, D), jnp.float32).at[:N].set(x)

    part1, pdeg = _sc_seg1(x_pad, srcs, dsts)
    q = _dense(x_pad, part1, pdeg.T, W1_self, W1_neigh, b1, W2_self, W2_neigh,
               b2)
    part2 = _sc_seg2(q, srcs, dsts)
    out = _combine(q, part2)
    return out[:N]
